# baseline (speedup 1.0000x reference)
import functools

import jax
import jax.numpy as jnp
from jax import lax
from jax.experimental import pallas as pl
from jax.experimental.pallas import tpu as pltpu
from jax.experimental.pallas import tpu_sc as plsc


_CAPACITY_RATIO = 0.75
_NC = 2
_NS = 16
_CHUNK = 512
_NBUF = 4


def _perm_body(s_ref, p_ref, *, k, l):
    s = s_ref[:]
    b = s.shape[0]
    bits = jax.lax.bitcast_convert_type(s, jnp.int32)
    key = jnp.where(bits >= 0, bits, bits ^ jnp.int32(0x7FFFFFFF))
    kk = jnp.int32(k)

    def cge(t):
        return jnp.sum((key >= t).astype(jnp.int32), axis=1, keepdims=True)

    zero = jnp.zeros((b, 1), jnp.int32)
    neg = jnp.full_like(zero, jnp.int32(-2147483648))
    t = jnp.where(cge(zero) >= kk, zero, neg)
    for bit in range(30, -1, -1):
        cand = t + jnp.int32(1 << bit)
        t = jnp.where(cge(cand) >= kk, cand, t)

    gt = key > t
    cnt_gt = jnp.sum(gt.astype(jnp.int32), axis=1, keepdims=True)
    need = kk - cnt_gt
    tie = key == t
    idx = jax.lax.broadcasted_iota(jnp.int32, s.shape, 1)
    bound = jnp.full_like(zero, jnp.int32(l))
    big = jnp.zeros_like(zero)
    for bit in range(12, -1, -1):
        cand = big + jnp.int32(1 << bit)
        cnt = jnp.sum((tie & (idx < cand)).astype(jnp.int32), axis=1,
                      keepdims=True)
        big = jnp.where((cand <= bound) & (cnt <= need), cand, big)

    sel = gt | (tie & (idx < big))

    ci = jax.lax.broadcasted_iota(jnp.int32, (_CHUNK, _CHUNK), 0)
    cj = jax.lax.broadcasted_iota(jnp.int32, (_CHUNK, _CHUNK), 1)
    t_strict = (ci < cj).astype(jnp.bfloat16)

    base = jnp.zeros((b, 1), jnp.int32)
    chunks = []
    for c in range(l // _CHUNK):
        m = sel[:, c * _CHUNK:(c + 1) * _CHUNK].astype(jnp.bfloat16)
        ex_in = jnp.dot(m, t_strict, preferred_element_type=jnp.float32)
        chunks.append(ex_in.astype(jnp.int32) + base)
        rowsum = jnp.sum(m.astype(jnp.float32), axis=1, keepdims=True)
        base = base + rowsum.astype(jnp.int32)
    srank = jnp.concatenate(chunks, axis=1)

    boff = jax.lax.broadcasted_iota(jnp.int32, s.shape, 0)
    urank = idx - srank
    p_ref[:] = jnp.where(
        sel,
        boff * jnp.int32(k) + srank,
        jnp.int32(b * k) + boff * jnp.int32(l - k) + urank)


def _permute_rows_body(x2d_hbm, perm3_hbm, out_hbm, idx_all,
                       rows0, rows1, rows2, rows3,
                       lsem0, lsem1, lsem2, lsem3,
                       ssem0, ssem1, ssem2, ssem3, isem,
                       *, rows_per_w, chunk):
    wid = lax.axis_index("s") * _NC + lax.axis_index("c")
    base = wid * rows_per_w
    nchunks = rows_per_w // chunk
    rows = (rows0, rows1, rows2, rows3)
    lsem = (lsem0, lsem1, lsem2, lsem3)
    ssem = (ssem0, ssem1, ssem2, ssem3)

    pltpu.async_copy(perm3_hbm.at[wid], idx_all, isem).wait()
    for bb in range(_NBUF):
        pltpu.async_copy(x2d_hbm.at[pl.ds(base + bb * chunk, chunk)],
                         rows[bb], lsem[bb])

    def body(g, _):
        for bb in range(_NBUF):
            cc = _NBUF * g + bb
            bb2 = (bb + _NBUF // 2) % _NBUF
            pltpu.make_async_copy(x2d_hbm.at[pl.ds(0, chunk)], rows[bb],
                                  lsem[bb]).wait()
            pltpu.async_copy(rows[bb], out_hbm.at[idx_all.at[cc]], ssem[bb])

            @pl.when(cc >= _NBUF // 2)
            def _():
                pltpu.make_async_copy(x2d_hbm.at[pl.ds(0, chunk)], rows[bb2],
                                      ssem[bb2]).wait()

                @pl.when(cc + _NBUF // 2 < nchunks)
                def _():
                    off2 = base + (cc + _NBUF // 2) * chunk
                    pltpu.async_copy(x2d_hbm.at[pl.ds(off2, chunk)],
                                     rows[bb2], lsem[bb2])
        return 0

    lax.fori_loop(0, nchunks // _NBUF, body, 0, unroll=False)
    for bb in range(_NBUF // 2, _NBUF):
        pltpu.make_async_copy(x2d_hbm.at[pl.ds(0, chunk)], rows[bb],
                              ssem[bb]).wait()


def _unpermute_rows_body(pool_hbm, perm3_hbm, out_hbm, idx_all,
                         rows0, rows1, rows2, rows3,
                         lsem0, lsem1, lsem2, lsem3,
                         ssem0, ssem1, ssem2, ssem3, isem,
                         *, rows_per_w, chunk):
    wid = lax.axis_index("s") * _NC + lax.axis_index("c")
    base = wid * rows_per_w
    nchunks = rows_per_w // chunk
    rows = (rows0, rows1, rows2, rows3)
    lsem = (lsem0, lsem1, lsem2, lsem3)
    ssem = (ssem0, ssem1, ssem2, ssem3)

    pltpu.async_copy(perm3_hbm.at[wid], idx_all, isem).wait()
    for bb in range(_NBUF):
        pltpu.async_copy(pool_hbm.at[idx_all.at[bb]], rows[bb], lsem[bb])

    def body(g, _):
        for bb in range(_NBUF):
            cc = _NBUF * g + bb
            bb2 = (bb + _NBUF // 2) % _NBUF
            pltpu.make_async_copy(pool_hbm.at[pl.ds(0, chunk)], rows[bb],
                                  lsem[bb]).wait()
            off = base + cc * chunk
            pltpu.async_copy(rows[bb], out_hbm.at[pl.ds(off, chunk)], ssem[bb])

            @pl.when(cc >= _NBUF // 2)
            def _():
                pltpu.make_async_copy(pool_hbm.at[pl.ds(0, chunk)], rows[bb2],
                                      ssem[bb2]).wait()

                @pl.when(cc + _NBUF // 2 < nchunks)
                def _():
                    pltpu.async_copy(
                        pool_hbm.at[idx_all.at[cc + _NBUF // 2]],
                        rows[bb2], lsem[bb2])
        return 0

    lax.fori_loop(0, nchunks // _NBUF, body, 0, unroll=False)
    for bb in range(_NBUF // 2, _NBUF):
        pltpu.make_async_copy(pool_hbm.at[pl.ds(0, chunk)], rows[bb],
                              ssem[bb]).wait()


def _mlp_dense_body(x_ref, w1_ref, w2_ref, o_ref):
    xb = x_ref[0]
    h = jnp.dot(xb.astype(jnp.bfloat16), w1_ref[:],
                preferred_element_type=jnp.float32)
    h = jnp.maximum(h, 0.0).astype(jnp.bfloat16)
    o_ref[0] = jnp.dot(h, w2_ref[:], preferred_element_type=jnp.float32)


def kernel(x, w_router, W1, W2):
    b, l, d = x.shape
    ff = W1.shape[1]
    k = max(1, int(l * _CAPACITY_RATIO))
    nw = _NC * _NS

    scores = jnp.einsum('bld,d->bl', x, w_router)

    perm = pl.pallas_call(
        functools.partial(_perm_body, k=k, l=l),
        out_shape=jax.ShapeDtypeStruct((b, l), jnp.int32),
    )(scores)

    x2d = x.reshape(b * l, d)
    rows_per_w = (b * l) // nw
    chunk = 8
    nchunks = rows_per_w // chunk
    perm3 = perm.reshape(nw, nchunks, chunk)

    sc_move = functools.partial(
        pl.kernel,
        mesh=plsc.VectorSubcoreMesh(core_axis_name="c", subcore_axis_name="s"),
        out_type=jax.ShapeDtypeStruct((b * l, d), jnp.float32),
        scratch_types=[
            pltpu.VMEM((nchunks, chunk), jnp.int32),
            pltpu.VMEM((chunk, d), jnp.float32),
            pltpu.VMEM((chunk, d), jnp.float32),
            pltpu.VMEM((chunk, d), jnp.float32),
            pltpu.VMEM((chunk, d), jnp.float32),
            pltpu.SemaphoreType.DMA,
            pltpu.SemaphoreType.DMA,
            pltpu.SemaphoreType.DMA,
            pltpu.SemaphoreType.DMA,
            pltpu.SemaphoreType.DMA,
            pltpu.SemaphoreType.DMA,
            pltpu.SemaphoreType.DMA,
            pltpu.SemaphoreType.DMA,
            pltpu.SemaphoreType.DMA,
        ],
    )
    pool = sc_move(functools.partial(
        _permute_rows_body, rows_per_w=rows_per_w, chunk=chunk))(
        x2d, perm3)

    lblk = 256
    nsel_blk = (b * k) // lblk
    npool_blk = (b * l) // lblk
    pool3 = pool.reshape(npool_blk, lblk, d)
    pool3 = pl.pallas_call(
        _mlp_dense_body,
        grid=(nsel_blk,),
        in_specs=[
            pl.BlockSpec((1, lblk, d), lambda i: (i, 0, 0)),
            pl.BlockSpec((d, ff), lambda i: (0, 0)),
            pl.BlockSpec((ff, d), lambda i: (0, 0)),
        ],
        out_specs=pl.BlockSpec((1, lblk, d), lambda i: (i, 0, 0)),
        out_shape=jax.ShapeDtypeStruct((npool_blk, lblk, d), jnp.float32),
        input_output_aliases={0: 0},
    )(pool3, W1.astype(jnp.bfloat16), W2.astype(jnp.bfloat16))

    out2d = sc_move(functools.partial(
        _unpermute_rows_body, rows_per_w=rows_per_w, chunk=chunk))(
        pool3.reshape(b * l, d), perm3)

    return out2d.reshape(b, l, d)


# speedup vs baseline: 1.0109x; 1.0109x over previous
import functools

import jax
import jax.numpy as jnp
from jax import lax
from jax.experimental import pallas as pl
from jax.experimental.pallas import tpu as pltpu
from jax.experimental.pallas import tpu_sc as plsc


_CAPACITY_RATIO = 0.75
_NC = 2
_NS = 16
_CHUNK = 512
_NBUF = 4


def _perm_body(s_ref, p_ref, *, k, l):
    s = s_ref[:]
    b = s.shape[0]
    bits = jax.lax.bitcast_convert_type(s, jnp.int32)
    key = jnp.where(bits >= 0, bits, bits ^ jnp.int32(0x7FFFFFFF))
    kk = jnp.int32(k)

    def cge(t):
        return jnp.sum((key >= t).astype(jnp.int32), axis=1, keepdims=True)

    zero = jnp.zeros((b, 1), jnp.int32)
    neg = jnp.full_like(zero, jnp.int32(-2147483648))
    t = jnp.where(cge(zero) >= kk, zero, neg)
    for bit in range(30, -1, -1):
        cand = t + jnp.int32(1 << bit)
        t = jnp.where(cge(cand) >= kk, cand, t)

    gt = key > t
    cnt_gt = jnp.sum(gt.astype(jnp.int32), axis=1, keepdims=True)
    need = kk - cnt_gt
    tie = key == t
    idx = jax.lax.broadcasted_iota(jnp.int32, s.shape, 1)
    bound = jnp.full_like(zero, jnp.int32(l))
    big = jnp.zeros_like(zero)
    for bit in range(12, -1, -1):
        cand = big + jnp.int32(1 << bit)
        cnt = jnp.sum((tie & (idx < cand)).astype(jnp.int32), axis=1,
                      keepdims=True)
        big = jnp.where((cand <= bound) & (cnt <= need), cand, big)

    sel = gt | (tie & (idx < big))

    ci = jax.lax.broadcasted_iota(jnp.int32, (_CHUNK, _CHUNK), 0)
    cj = jax.lax.broadcasted_iota(jnp.int32, (_CHUNK, _CHUNK), 1)
    t_strict = (ci < cj).astype(jnp.bfloat16)

    base = jnp.zeros((b, 1), jnp.int32)
    chunks = []
    for c in range(l // _CHUNK):
        m = sel[:, c * _CHUNK:(c + 1) * _CHUNK].astype(jnp.bfloat16)
        ex_in = jnp.dot(m, t_strict, preferred_element_type=jnp.float32)
        chunks.append(ex_in.astype(jnp.int32) + base)
        rowsum = jnp.sum(m.astype(jnp.float32), axis=1, keepdims=True)
        base = base + rowsum.astype(jnp.int32)
    srank = jnp.concatenate(chunks, axis=1)

    boff = jax.lax.broadcasted_iota(jnp.int32, s.shape, 0)
    urank = idx - srank
    p_ref[:] = jnp.where(
        sel,
        boff * jnp.int32(k) + srank,
        jnp.int32(b * k) + boff * jnp.int32(l - k) + urank)


def _permute_rows_body(x2d_hbm, perm3_hbm, out_hbm, idx_all,
                       rows0, rows1, rows2, rows3,
                       lsem0, lsem1, lsem2, lsem3,
                       ssem0, ssem1, ssem2, ssem3, isem,
                       *, rows_per_w, chunk):
    wid = lax.axis_index("s") * _NC + lax.axis_index("c")
    base = wid * rows_per_w
    nchunks = rows_per_w // chunk
    rows = (rows0, rows1, rows2, rows3)
    lsem = (lsem0, lsem1, lsem2, lsem3)
    ssem = (ssem0, ssem1, ssem2, ssem3)

    pltpu.async_copy(perm3_hbm.at[wid], idx_all, isem).wait()
    for bb in range(_NBUF):
        pltpu.async_copy(x2d_hbm.at[pl.ds(base + bb * chunk, chunk)],
                         rows[bb], lsem[bb])

    def body(g, _):
        for bb in range(_NBUF):
            cc = _NBUF * g + bb
            bb2 = (bb + _NBUF // 2) % _NBUF
            pltpu.make_async_copy(x2d_hbm.at[pl.ds(0, chunk)], rows[bb],
                                  lsem[bb]).wait()
            pltpu.async_copy(rows[bb], out_hbm.at[idx_all.at[cc]], ssem[bb])

            @pl.when(cc >= _NBUF // 2)
            def _():
                pltpu.make_async_copy(x2d_hbm.at[pl.ds(0, chunk)], rows[bb2],
                                      ssem[bb2]).wait()

                @pl.when(cc + _NBUF // 2 < nchunks)
                def _():
                    off2 = base + (cc + _NBUF // 2) * chunk
                    pltpu.async_copy(x2d_hbm.at[pl.ds(off2, chunk)],
                                     rows[bb2], lsem[bb2])
        return 0

    lax.fori_loop(0, nchunks // _NBUF, body, 0, unroll=False)
    for bb in range(_NBUF // 2, _NBUF):
        pltpu.make_async_copy(x2d_hbm.at[pl.ds(0, chunk)], rows[bb],
                              ssem[bb]).wait()


def _unpermute_rows_body(pool_hbm, perm3_hbm, out_hbm, idx_all,
                         rows0, rows1, rows2, rows3,
                         lsem0, lsem1, lsem2, lsem3,
                         ssem0, ssem1, ssem2, ssem3, isem,
                         *, rows_per_w, chunk):
    wid = lax.axis_index("s") * _NC + lax.axis_index("c")
    base = wid * rows_per_w
    nchunks = rows_per_w // chunk
    rows = (rows0, rows1, rows2, rows3)
    lsem = (lsem0, lsem1, lsem2, lsem3)
    ssem = (ssem0, ssem1, ssem2, ssem3)

    pltpu.async_copy(perm3_hbm.at[wid], idx_all, isem).wait()
    for bb in range(_NBUF):
        pltpu.async_copy(pool_hbm.at[idx_all.at[bb]], rows[bb], lsem[bb])

    def body(g, _):
        for bb in range(_NBUF):
            cc = _NBUF * g + bb
            bb2 = (bb + _NBUF // 2) % _NBUF
            pltpu.make_async_copy(pool_hbm.at[pl.ds(0, chunk)], rows[bb],
                                  lsem[bb]).wait()
            off = base + cc * chunk
            pltpu.async_copy(rows[bb], out_hbm.at[pl.ds(off, chunk)], ssem[bb])

            @pl.when(cc >= _NBUF // 2)
            def _():
                pltpu.make_async_copy(pool_hbm.at[pl.ds(0, chunk)], rows[bb2],
                                      ssem[bb2]).wait()

                @pl.when(cc + _NBUF // 2 < nchunks)
                def _():
                    pltpu.async_copy(
                        pool_hbm.at[idx_all.at[cc + _NBUF // 2]],
                        rows[bb2], lsem[bb2])
        return 0

    lax.fori_loop(0, nchunks // _NBUF, body, 0, unroll=False)
    for bb in range(_NBUF // 2, _NBUF):
        pltpu.make_async_copy(pool_hbm.at[pl.ds(0, chunk)], rows[bb],
                              ssem[bb]).wait()


def _mlp_dense_body(x_ref, w1_ref, w2_ref, o_ref):
    xb = x_ref[0]
    h = jnp.dot(xb.astype(jnp.bfloat16), w1_ref[:],
                preferred_element_type=jnp.float32)
    h = jnp.maximum(h, 0.0).astype(jnp.bfloat16)
    o_ref[0] = jnp.dot(h, w2_ref[:], preferred_element_type=jnp.float32)


def kernel(x, w_router, W1, W2):
    b, l, d = x.shape
    ff = W1.shape[1]
    k = max(1, int(l * _CAPACITY_RATIO))
    nw = _NC * _NS

    scores = jnp.einsum('bld,d->bl', x, w_router)

    perm = pl.pallas_call(
        functools.partial(_perm_body, k=k, l=l),
        out_shape=jax.ShapeDtypeStruct((b, l), jnp.int32),
    )(scores)

    x2d = x.reshape(b * l, d)
    rows_per_w = (b * l) // nw
    chunk = 8
    nchunks = rows_per_w // chunk
    perm3 = perm.reshape(nw, nchunks, chunk)

    sc_move = functools.partial(
        pl.kernel,
        mesh=plsc.VectorSubcoreMesh(core_axis_name="c", subcore_axis_name="s"),
        out_type=jax.ShapeDtypeStruct((b * l, d), jnp.float32),
        scratch_types=[
            pltpu.VMEM((nchunks, chunk), jnp.int32),
            pltpu.VMEM((chunk, d), jnp.float32),
            pltpu.VMEM((chunk, d), jnp.float32),
            pltpu.VMEM((chunk, d), jnp.float32),
            pltpu.VMEM((chunk, d), jnp.float32),
            pltpu.SemaphoreType.DMA,
            pltpu.SemaphoreType.DMA,
            pltpu.SemaphoreType.DMA,
            pltpu.SemaphoreType.DMA,
            pltpu.SemaphoreType.DMA,
            pltpu.SemaphoreType.DMA,
            pltpu.SemaphoreType.DMA,
            pltpu.SemaphoreType.DMA,
            pltpu.SemaphoreType.DMA,
        ],
    )
    pool = sc_move(functools.partial(
        _permute_rows_body, rows_per_w=rows_per_w, chunk=chunk))(
        x2d, perm3)

    lblk = 512
    nsel_blk = (b * k) // lblk
    npool_blk = (b * l) // lblk
    pool3 = pool.reshape(npool_blk, lblk, d)
    pool3 = pl.pallas_call(
        _mlp_dense_body,
        grid=(nsel_blk,),
        in_specs=[
            pl.BlockSpec((1, lblk, d), lambda i: (i, 0, 0)),
            pl.BlockSpec((d, ff), lambda i: (0, 0)),
            pl.BlockSpec((ff, d), lambda i: (0, 0)),
        ],
        out_specs=pl.BlockSpec((1, lblk, d), lambda i: (i, 0, 0)),
        out_shape=jax.ShapeDtypeStruct((npool_blk, lblk, d), jnp.float32),
        input_output_aliases={0: 0},
    )(pool3, W1.astype(jnp.bfloat16), W2.astype(jnp.bfloat16))

    out2d = sc_move(functools.partial(
        _unpermute_rows_body, rows_per_w=rows_per_w, chunk=chunk))(
        pool3.reshape(b * l, d), perm3)

    return out2d.reshape(b, l, d)
